# baseline (speedup 1.0000x reference)
import jax
import jax.numpy as jnp
from jax import lax
from jax.experimental import pallas as pl
from jax.experimental.pallas import tpu as pltpu
from jax.experimental.pallas import tpu_sc as plsc

ANG = 0.52917721067
INV_ANG = 1.0 / ANG
SQRT3 = 1.7320508075688772

N_NODES = 100000
NPAD = 102400
N_EDGES = 3200000
ROW = 128
K = 4
NWORK = 32
NROWS = 25600
EPAD = NROWS * ROW
CHUNK = K * ROW
NZ = 95
NREF = 5
RC6_ROWS = NZ * NZ
RC6_PAD = 9088
RC6_W = 32

ROWS_PER_W = NROWS // NWORK
NBATCH = ROWS_PER_W // K
NSLICE = NPAD // 16
BLK = 2048


def _zero_fill(ref, n):
  @pl.loop(0, n // 16)
  def _(i):
    ref[pl.ds(i * 16, 16)] = jnp.zeros((16,), jnp.float32)


def _sc_mesh():
  return plsc.VectorSubcoreMesh(core_axis_name="c", subcore_axis_name="s")


def _col(c):
  return jnp.full((16,), c, jnp.int32)


def _pass1_body(sp_hbm, ein_hbm, cov_hbm, out_hbm,
                cn_sh, cov_v,
                ein0_v, ein1_v, sps0_v, sps1_v, spd0_v, spd1_v,
                cn0_v, cn1_v, ob_v,
                sem_lin, sem_g, sem_s):
  cid = lax.axis_index("c")
  sid = lax.axis_index("s")
  wid = sid * 2 + cid

  sl_stage = pl.ds(sid * NSLICE, NSLICE)
  _zero_fill(ob_v, NSLICE)
  pltpu.sync_copy(ob_v, cn_sh.at[sl_stage])
  pltpu.sync_copy(cov_hbm, cov_v)
  plsc.subcore_barrier()

  r0 = wid * ROWS_PER_W
  ein = (ein0_v, ein1_v)
  sps = (sps0_v, sps1_v)
  spd = (spd0_v, spd1_v)
  cnb = (cn0_v, cn1_v)

  def lin_desc(b, s):
    return pltpu.make_async_copy(ein_hbm.at[pl.ds(r0 + b * K, K)],
                                 ein[s], sem_lin)

  def issue_gathers(s):
    for k in range(K):
      dst = pl.ds(k * ROW, ROW)
      pltpu.make_async_copy(sp_hbm.at[ein[s].at[k, 0]],
                            sps[s].at[dst], sem_g).start()
      pltpu.make_async_copy(sp_hbm.at[ein[s].at[k, 1]],
                            spd[s].at[dst], sem_g).start()

  def wait_gathers(s):
    pltpu.make_async_copy(sp_hbm.at[pl.ds(0, CHUNK)], sps[s], sem_g).wait()
    pltpu.make_async_copy(sp_hbm.at[pl.ds(0, CHUNK)], spd[s], sem_g).wait()

  def issue_scatters(s):
    for k in range(K):
      pltpu.make_async_copy(cnb[s].at[pl.ds(k * ROW, ROW)],
                            cn_sh.at[ein[s].at[k, 0]],
                            sem_s).start(add=True)

  def wait_scatters(s):
    pltpu.make_async_copy(out_hbm.at[cid, pl.ds(0, CHUNK)], cnb[s],
                          sem_s).wait()

  def compute(s):
    for g in range(CHUNK // 16):
      sl = pl.ds(g * 16, 16)
      k = g // (ROW // 16)
      gi = g % (ROW // 16)
      rc = (plsc.load_gather(cov_v, [sps[s][sl]]) +
            plsc.load_gather(cov_v, [spd[s][sl]]))
      dist = plsc.bitcast(ein[s][k, 2, pl.ds(gi * 16, 16)], jnp.float32)
      rij = jnp.maximum(dist * INV_ANG, 1e-6)
      x = 16.0 * (rc / rij - 1.0)
      cnb[s][sl] = 1.0 / (1.0 + jnp.exp(-x))

  def batch(j, b, s, first, last):
    wait_gathers(s)
    if first:
      @pl.when(j > 0)
      def _():
        wait_scatters(1 - s)
    else:
      wait_scatters(1 - s)

    def prefetch():
      lin_desc(b + 1, 1 - s).start()
      lin_desc(b + 1, 1 - s).wait()
      issue_gathers(1 - s)
    if last:
      @pl.when(j < (NBATCH // 2 - 1))
      def _():
        prefetch()
    else:
      prefetch()
    compute(s)
    issue_scatters(s)

  lin_desc(0, 0).start()
  lin_desc(0, 0).wait()
  issue_gathers(0)

  @pl.loop(0, NBATCH // 2)
  def _(j):
    batch(j, j * 2, 0, True, False)
    batch(j, j * 2 + 1, 1, False, True)

  wait_scatters(1)

  plsc.subcore_barrier()
  pltpu.sync_copy(cn_sh.at[sl_stage], ob_v)
  pltpu.sync_copy(ob_v, out_hbm.at[cid, sl_stage])


def _run_pass1(sp_p, ein, cov_p):
  fn = pl.kernel(
      _pass1_body,
      out_type=jax.ShapeDtypeStruct((2, NPAD), jnp.float32),
      mesh=_sc_mesh(),
      compiler_params=pltpu.CompilerParams(
          needs_layout_passes=False, use_tc_tiling_on_sc=False),
      scratch_types=[
          pltpu.VMEM_SHARED((NPAD,), jnp.float32),
          pltpu.VMEM((96,), jnp.float32),
          pltpu.VMEM((K, 4, ROW), jnp.int32),
          pltpu.VMEM((K, 4, ROW), jnp.int32),
          pltpu.VMEM((CHUNK,), jnp.int32),
          pltpu.VMEM((CHUNK,), jnp.int32),
          pltpu.VMEM((CHUNK,), jnp.int32),
          pltpu.VMEM((CHUNK,), jnp.int32),
          pltpu.VMEM((CHUNK,), jnp.float32),
          pltpu.VMEM((CHUNK,), jnp.float32),
          pltpu.VMEM((NSLICE,), jnp.float32),
          pltpu.SemaphoreType.DMA,
          pltpu.SemaphoreType.DMA,
          pltpu.SemaphoreType.DMA,
      ],
  )
  return fn(sp_p, ein, cov_p)


def _node_body(sp_ref, p0_ref, p1_ref, t_ref, nd_ref):
  sp = sp_ref[...]
  oh = (sp[:, None] == lax.broadcasted_iota(jnp.int32, (BLK, 128), 1)
        ).astype(jnp.float32)
  r = jnp.dot(oh, t_ref[...], preferred_element_type=jnp.float32)
  refcn = r[:, 0:NREF]
  exw = r[:, NREF:2 * NREF]
  g = r[:, 2 * NREF:2 * NREF + 1]
  cn = p0_ref[...] + p1_ref[...]
  mask = refcn >= 0.0
  dcn = refcn - cn[:, None]
  w = jnp.where(mask, jnp.exp(-4.0 * dcn * dcn), 0.0)
  norm = jnp.sum(w, axis=1, keepdims=True)
  wn = jnp.where(mask, w / jnp.maximum(norm, 1e-6), 0.0)
  wf = jnp.where(norm < 1e-6, exw, wn)
  spf = lax.bitcast_convert_type(sp, jnp.float32)[:, None]
  nd_ref[...] = jnp.concatenate(
      [wf, g, spf, jnp.zeros((BLK, 1), jnp.float32)], axis=1)


def _run_node(sp_p, p0, p1, table):
  return pl.pallas_call(
      _node_body,
      grid=(NPAD // BLK,),
      in_specs=[
          pl.BlockSpec((BLK,), lambda i: (i,)),
          pl.BlockSpec((BLK,), lambda i: (i,)),
          pl.BlockSpec((BLK,), lambda i: (i,)),
          pl.BlockSpec((128, 128), lambda i: (0, 0)),
      ],
      out_specs=pl.BlockSpec((BLK, 8), lambda i: (i, 0)),
      out_shape=jax.ShapeDtypeStruct((NPAD, 8), jnp.float32),
  )(sp_p, p0, p1, table)


def _pass2_body(ein_hbm, nd_hbm, rc6_hbm, out_hbm,
                e_sh,
                ein0_v, ein1_v, nds0_v, nds1_v, ndd0_v, ndd1_v,
                pair_v, rc6_v, e0_v, e1_v, ob_v,
                sem_lin, sem_g, sem_r, sem_s):
  cid = lax.axis_index("c")
  sid = lax.axis_index("s")
  wid = sid * 2 + cid

  sl_stage = pl.ds(sid * NSLICE, NSLICE)
  _zero_fill(ob_v, NSLICE)
  pltpu.sync_copy(ob_v, e_sh.at[sl_stage])
  plsc.subcore_barrier()

  r0 = wid * ROWS_PER_W
  ein = (ein0_v, ein1_v)
  nds = (nds0_v, nds1_v)
  ndd = (ndd0_v, ndd1_v)
  ev = (e0_v, e1_v)

  def lin_desc(b, s):
    return pltpu.make_async_copy(ein_hbm.at[pl.ds(r0 + b * K, K)],
                                 ein[s], sem_lin)

  def issue_gathers(s):
    for k in range(K):
      dst = pl.ds(k * ROW, ROW)
      pltpu.make_async_copy(nd_hbm.at[ein[s].at[k, 0]],
                            nds[s].at[dst], sem_g).start()
      pltpu.make_async_copy(nd_hbm.at[ein[s].at[k, 1]],
                            ndd[s].at[dst], sem_g).start()

  def wait_gathers(s):
    pltpu.make_async_copy(nd_hbm.at[pl.ds(0, CHUNK)], nds[s], sem_g).wait()
    pltpu.make_async_copy(nd_hbm.at[pl.ds(0, CHUNK)], ndd[s], sem_g).wait()

  def issue_rc6():
    for k in range(K):
      pltpu.make_async_copy(rc6_hbm.at[pair_v.at[k]],
                            rc6_v.at[pl.ds(k * ROW, ROW)], sem_r).start()

  def wait_rc6():
    pltpu.make_async_copy(rc6_hbm.at[pl.ds(0, CHUNK)], rc6_v, sem_r).wait()

  def issue_scatters(s):
    for k in range(K):
      pltpu.make_async_copy(ev[s].at[pl.ds(k * ROW, ROW)],
                            e_sh.at[ein[s].at[k, 0]],
                            sem_s).start(add=True)

  def wait_scatters(s):
    pltpu.make_async_copy(out_hbm.at[cid, pl.ds(0, CHUNK)], ev[s],
                          sem_s).wait()

  def pair_compute(s):
    for g in range(CHUNK // 16):
      sl = pl.ds(g * 16, 16)
      r16 = lax.iota(jnp.int32, 16) + (g * 16)
      sp_s = plsc.bitcast(
          plsc.load_gather(nds[s], [r16, _col(6)]), jnp.int32)
      sp_d = plsc.bitcast(
          plsc.load_gather(ndd[s], [r16, _col(6)]), jnp.int32)
      pair_v[g // (ROW // 16), pl.ds((g % (ROW // 16)) * 16, 16)] = (
          sp_s * NZ + sp_d)

  def compute(s):
    for g in range(CHUNK // 16):
      sl = pl.ds(g * 16, 16)
      r16 = lax.iota(jnp.int32, 16) + (g * 16)
      k = g // (ROW // 16)
      gi = g % (ROW // 16)
      ws = [plsc.load_gather(nds[s], [r16, _col(a)]) for a in range(NREF)]
      wd = [plsc.load_gather(ndd[s], [r16, _col(b)]) for b in range(NREF)]
      gs16 = plsc.load_gather(nds[s], [r16, _col(NREF)])
      gd16 = plsc.load_gather(ndd[s], [r16, _col(NREF)])
      c6 = jnp.zeros((16,), jnp.float32)
      for a in range(NREF):
        for b in range(NREF):
          cab = plsc.load_gather(rc6_v, [r16, _col(a * NREF + b)])
          c6 = c6 + cab * ws[a] * wd[b]
      gg = gs16 * gd16
      qq = 3.0 * gg * gg
      r0d = (0.4 * SQRT3) * gg + 5.0
      dist = plsc.bitcast(ein[s][k, 2, pl.ds(gi * 16, 16)], jnp.float32)
      sw = plsc.bitcast(ein[s][k, 3, pl.ds(gi * 16, 16)], jnp.float32)
      rij = jnp.maximum(dist * INV_ANG, 1e-6)
      r2 = rij * rij
      r4 = r2 * r2
      r6 = r4 * r2
      r8 = r4 * r4
      p2 = r0d * r0d
      p4 = p2 * p2
      p6 = p4 * p2
      p8 = p4 * p4
      t6 = 1.0 / (r6 + p6)
      t8 = 1.0 / (r8 + p8)
      ev[s][sl] = (-0.5) * sw * (c6 * t6 + c6 * qq * t8)

  def batch(j, b, s, first, last):
    wait_gathers(s)
    pair_compute(s)
    issue_rc6()
    if first:
      @pl.when(j > 0)
      def _():
        wait_scatters(1 - s)
    else:
      wait_scatters(1 - s)

    def prefetch():
      lin_desc(b + 1, 1 - s).start()
      lin_desc(b + 1, 1 - s).wait()
      issue_gathers(1 - s)
    if last:
      @pl.when(j < (NBATCH // 2 - 1))
      def _():
        prefetch()
    else:
      prefetch()
    wait_rc6()
    compute(s)
    issue_scatters(s)

  lin_desc(0, 0).start()
  lin_desc(0, 0).wait()
  issue_gathers(0)

  @pl.loop(0, NBATCH // 2)
  def _(j):
    batch(j, j * 2, 0, True, False)
    batch(j, j * 2 + 1, 1, False, True)

  wait_scatters(1)

  plsc.subcore_barrier()
  pltpu.sync_copy(e_sh.at[sl_stage], ob_v)
  pltpu.sync_copy(ob_v, out_hbm.at[cid, sl_stage])


def _run_pass2(ein, nd, rc6p):
  fn = pl.kernel(
      _pass2_body,
      out_type=jax.ShapeDtypeStruct((2, NPAD), jnp.float32),
      mesh=_sc_mesh(),
      compiler_params=pltpu.CompilerParams(
          needs_layout_passes=False, use_tc_tiling_on_sc=False),
      scratch_types=[
          pltpu.VMEM_SHARED((NPAD,), jnp.float32),
          pltpu.VMEM((K, 4, ROW), jnp.int32),
          pltpu.VMEM((K, 4, ROW), jnp.int32),
          pltpu.VMEM((CHUNK, 8), jnp.float32),
          pltpu.VMEM((CHUNK, 8), jnp.float32),
          pltpu.VMEM((CHUNK, 8), jnp.float32),
          pltpu.VMEM((CHUNK, 8), jnp.float32),
          pltpu.VMEM((K, ROW), jnp.int32),
          pltpu.VMEM((CHUNK, RC6_W), jnp.float32),
          pltpu.VMEM((CHUNK,), jnp.float32),
          pltpu.VMEM((CHUNK,), jnp.float32),
          pltpu.VMEM((NSLICE,), jnp.float32),
          pltpu.SemaphoreType.DMA,
          pltpu.SemaphoreType.DMA,
          pltpu.SemaphoreType.DMA,
          pltpu.SemaphoreType.DMA,
      ],
  )
  return fn(ein, nd, rc6p)


def _final_body(e0_ref, e1_ref, out_ref):
  out_ref[...] = e0_ref[...] + e1_ref[...]


def _run_final(e0, e1):
  return pl.pallas_call(
      _final_body,
      grid=(NPAD // BLK,),
      in_specs=[
          pl.BlockSpec((BLK,), lambda i: (i,)),
          pl.BlockSpec((BLK,), lambda i: (i,)),
      ],
      out_specs=pl.BlockSpec((BLK,), lambda i: (i,)),
      out_shape=jax.ShapeDtypeStruct((NPAD,), jnp.float32),
  )(e0, e1)


@jax.jit
def kernel(species, edge_src, edge_dst, distances, switch,
           cov_d3, r4r2, ref_cn, ref_c6):
  sp_p = jnp.zeros((NPAD,), jnp.int32).at[:N_NODES].set(species)

  npad_extra = EPAD - N_EDGES
  pad_idx = (jnp.arange(npad_extra, dtype=jnp.int32) % (NPAD - N_NODES)
             ) + N_NODES
  es_p = jnp.concatenate([edge_src, pad_idx])
  ed_p = jnp.concatenate([edge_dst, pad_idx])
  d_p = jnp.concatenate(
      [distances, jnp.ones((npad_extra,), jnp.float32)])
  sw_p = jnp.concatenate(
      [switch, jnp.zeros((npad_extra,), jnp.float32)])
  ein = jnp.stack([
      es_p.reshape(NROWS, ROW),
      ed_p.reshape(NROWS, ROW),
      lax.bitcast_convert_type(d_p, jnp.int32).reshape(NROWS, ROW),
      lax.bitcast_convert_type(sw_p, jnp.int32).reshape(NROWS, ROW),
  ], axis=1)
  cov_p = jnp.zeros((96,), jnp.float32).at[:NZ].set(cov_d3)

  g = jnp.sqrt(r4r2)
  exw = jax.nn.one_hot(jnp.argmax(ref_cn, axis=1), NREF, dtype=jnp.float32)
  table = jnp.zeros((128, 128), jnp.float32)
  table = table.at[:NZ, 0:NREF].set(ref_cn)
  table = table.at[:NZ, NREF:2 * NREF].set(exw)
  table = table.at[:NZ, 2 * NREF].set(g)

  rc6p = jnp.zeros((RC6_PAD, RC6_W), jnp.float32)
  rc6p = rc6p.at[:RC6_ROWS, :NREF * NREF].set(
      ref_c6.reshape(RC6_ROWS, NREF * NREF))

  cnp = _run_pass1(sp_p, ein, cov_p)
  nd = _run_node(sp_p, cnp[0], cnp[1], table)
  ep = _run_pass2(ein, nd, rc6p)
  energy = _run_final(ep[0], ep[1])
  return energy[:N_NODES]


# speedup vs baseline: 1.1983x; 1.1983x over previous
import jax
import jax.numpy as jnp
from jax import lax
from jax.experimental import pallas as pl
from jax.experimental.pallas import tpu as pltpu
from jax.experimental.pallas import tpu_sc as plsc

ANG = 0.52917721067
INV_ANG = 1.0 / ANG
SQRT3 = 1.7320508075688772

N_NODES = 100000
NPAD = 102400
N_EDGES = 3200000
ROW = 80
K = 5
NWORK = 32
NROWS = 40000
EPAD = NROWS * ROW
CHUNK = K * ROW
NZ = 95
NREF = 5
RC6_ROWS = NZ * NZ
RC6_PAD = 9088
RC6_W = 32

ROWS_PER_W = NROWS // NWORK
NBATCH = ROWS_PER_W // K
NSLICE = NPAD // 16
BLK = 2048


def _zero_fill(ref, n):
  @pl.loop(0, n // 16)
  def _(i):
    ref[pl.ds(i * 16, 16)] = jnp.zeros((16,), jnp.float32)


def _sc_mesh():
  return plsc.VectorSubcoreMesh(core_axis_name="c", subcore_axis_name="s")


def _col(c):
  return jnp.full((16,), c, jnp.int32)


def _pass1_body(sp_hbm, ein_hbm, cov_hbm, out_hbm,
                cn_sh, cov_v,
                ein0_v, ein1_v, sps0_v, sps1_v, spd0_v, spd1_v,
                cn0_v, cn1_v, ob_v,
                sem_lin, sem_g, sem_s):
  cid = lax.axis_index("c")
  sid = lax.axis_index("s")
  wid = sid * 2 + cid

  sl_stage = pl.ds(sid * NSLICE, NSLICE)
  _zero_fill(ob_v, NSLICE)
  pltpu.sync_copy(ob_v, cn_sh.at[sl_stage])
  pltpu.sync_copy(cov_hbm, cov_v)
  plsc.subcore_barrier()

  r0 = wid * ROWS_PER_W
  ein = (ein0_v, ein1_v)
  sps = (sps0_v, sps1_v)
  spd = (spd0_v, spd1_v)
  cnb = (cn0_v, cn1_v)

  def lin_desc(b, s):
    return pltpu.make_async_copy(ein_hbm.at[pl.ds(r0 + b * K, K)],
                                 ein[s], sem_lin)

  def issue_gathers(s):
    for k in range(K):
      dst = pl.ds(k * ROW, ROW)
      pltpu.make_async_copy(sp_hbm.at[ein[s].at[k, 0]],
                            sps[s].at[dst], sem_g).start()
      pltpu.make_async_copy(sp_hbm.at[ein[s].at[k, 1]],
                            spd[s].at[dst], sem_g).start()

  def wait_gathers(s):
    pltpu.make_async_copy(sp_hbm.at[pl.ds(0, CHUNK)], sps[s], sem_g).wait()
    pltpu.make_async_copy(sp_hbm.at[pl.ds(0, CHUNK)], spd[s], sem_g).wait()

  def issue_scatters(s):
    for k in range(K):
      pltpu.make_async_copy(cnb[s].at[pl.ds(k * ROW, ROW)],
                            cn_sh.at[ein[s].at[k, 0]],
                            sem_s).start(add=True)

  def wait_scatters(s):
    pltpu.make_async_copy(out_hbm.at[cid, pl.ds(0, CHUNK)], cnb[s],
                          sem_s).wait()

  def compute(s):
    for g in range(CHUNK // 16):
      sl = pl.ds(g * 16, 16)
      k = g // (ROW // 16)
      gi = g % (ROW // 16)
      rc = (plsc.load_gather(cov_v, [sps[s][sl]]) +
            plsc.load_gather(cov_v, [spd[s][sl]]))
      dist = plsc.bitcast(ein[s][k, 2, pl.ds(gi * 16, 16)], jnp.float32)
      rij = jnp.maximum(dist * INV_ANG, 1e-6)
      x = 16.0 * (rc / rij - 1.0)
      cnb[s][sl] = 1.0 / (1.0 + jnp.exp(-x))

  def batch(j, b, s, first, last):
    wait_gathers(s)
    if first:
      @pl.when(j > 0)
      def _():
        wait_scatters(1 - s)
    else:
      wait_scatters(1 - s)

    def prefetch():
      lin_desc(b + 1, 1 - s).start()
      lin_desc(b + 1, 1 - s).wait()
      issue_gathers(1 - s)
    if last:
      @pl.when(j < (NBATCH // 2 - 1))
      def _():
        prefetch()
    else:
      prefetch()
    compute(s)
    issue_scatters(s)

  lin_desc(0, 0).start()
  lin_desc(0, 0).wait()
  issue_gathers(0)

  @pl.loop(0, NBATCH // 2)
  def _(j):
    batch(j, j * 2, 0, True, False)
    batch(j, j * 2 + 1, 1, False, True)

  wait_scatters(1)

  plsc.subcore_barrier()
  pltpu.sync_copy(cn_sh.at[sl_stage], ob_v)
  pltpu.sync_copy(ob_v, out_hbm.at[cid, sl_stage])


def _run_pass1(sp_p, ein, cov_p):
  fn = pl.kernel(
      _pass1_body,
      out_type=jax.ShapeDtypeStruct((2, NPAD), jnp.float32),
      mesh=_sc_mesh(),
      compiler_params=pltpu.CompilerParams(
          needs_layout_passes=False, use_tc_tiling_on_sc=False),
      scratch_types=[
          pltpu.VMEM_SHARED((NPAD,), jnp.float32),
          pltpu.VMEM((96,), jnp.float32),
          pltpu.VMEM((K, 4, ROW), jnp.int32),
          pltpu.VMEM((K, 4, ROW), jnp.int32),
          pltpu.VMEM((CHUNK,), jnp.int32),
          pltpu.VMEM((CHUNK,), jnp.int32),
          pltpu.VMEM((CHUNK,), jnp.int32),
          pltpu.VMEM((CHUNK,), jnp.int32),
          pltpu.VMEM((CHUNK,), jnp.float32),
          pltpu.VMEM((CHUNK,), jnp.float32),
          pltpu.VMEM((NSLICE,), jnp.float32),
          pltpu.SemaphoreType.DMA,
          pltpu.SemaphoreType.DMA,
          pltpu.SemaphoreType.DMA,
      ],
  )
  return fn(sp_p, ein, cov_p)


def _node_body(sp_ref, p0_ref, p1_ref, t_ref, nd_ref):
  sp = sp_ref[...]
  oh = (sp[:, None] == lax.broadcasted_iota(jnp.int32, (BLK, 128), 1)
        ).astype(jnp.float32)
  r = jnp.dot(oh, t_ref[...], preferred_element_type=jnp.float32)
  refcn = r[:, 0:NREF]
  exw = r[:, NREF:2 * NREF]
  g = r[:, 2 * NREF:2 * NREF + 1]
  cn = p0_ref[...] + p1_ref[...]
  mask = refcn >= 0.0
  dcn = refcn - cn[:, None]
  w = jnp.where(mask, jnp.exp(-4.0 * dcn * dcn), 0.0)
  norm = jnp.sum(w, axis=1, keepdims=True)
  wn = jnp.where(mask, w / jnp.maximum(norm, 1e-6), 0.0)
  wf = jnp.where(norm < 1e-6, exw, wn)
  spf = lax.bitcast_convert_type(sp, jnp.float32)[:, None]
  nd_ref[...] = jnp.concatenate(
      [wf, g, spf, jnp.zeros((BLK, 1), jnp.float32)], axis=1)


def _run_node(sp_p, p0, p1, table):
  return pl.pallas_call(
      _node_body,
      grid=(NPAD // BLK,),
      in_specs=[
          pl.BlockSpec((BLK,), lambda i: (i,)),
          pl.BlockSpec((BLK,), lambda i: (i,)),
          pl.BlockSpec((BLK,), lambda i: (i,)),
          pl.BlockSpec((128, 128), lambda i: (0, 0)),
      ],
      out_specs=pl.BlockSpec((BLK, 8), lambda i: (i, 0)),
      out_shape=jax.ShapeDtypeStruct((NPAD, 8), jnp.float32),
  )(sp_p, p0, p1, table)


def _pass2_body(ein_hbm, nd_hbm, rc6_hbm, out_hbm,
                e_sh,
                ein0_v, ein1_v, nds0_v, nds1_v, ndd0_v, ndd1_v,
                pair_v, rc6_v, e0_v, e1_v, ob_v,
                sem_lin, sem_g, sem_r, sem_s):
  cid = lax.axis_index("c")
  sid = lax.axis_index("s")
  wid = sid * 2 + cid

  sl_stage = pl.ds(sid * NSLICE, NSLICE)
  _zero_fill(ob_v, NSLICE)
  pltpu.sync_copy(ob_v, e_sh.at[sl_stage])
  plsc.subcore_barrier()

  r0 = wid * ROWS_PER_W
  ein = (ein0_v, ein1_v)
  nds = (nds0_v, nds1_v)
  ndd = (ndd0_v, ndd1_v)
  ev = (e0_v, e1_v)

  def lin_desc(b, s):
    return pltpu.make_async_copy(ein_hbm.at[pl.ds(r0 + b * K, K)],
                                 ein[s], sem_lin)

  def issue_gathers(s):
    for k in range(K):
      dst = pl.ds(k * ROW, ROW)
      pltpu.make_async_copy(nd_hbm.at[ein[s].at[k, 0]],
                            nds[s].at[dst], sem_g).start()
      pltpu.make_async_copy(nd_hbm.at[ein[s].at[k, 1]],
                            ndd[s].at[dst], sem_g).start()

  def wait_gathers(s):
    pltpu.make_async_copy(nd_hbm.at[pl.ds(0, CHUNK)], nds[s], sem_g).wait()
    pltpu.make_async_copy(nd_hbm.at[pl.ds(0, CHUNK)], ndd[s], sem_g).wait()

  def issue_rc6():
    for k in range(K):
      pltpu.make_async_copy(rc6_hbm.at[pair_v.at[k]],
                            rc6_v.at[pl.ds(k * ROW, ROW)], sem_r).start()

  def wait_rc6():
    pltpu.make_async_copy(rc6_hbm.at[pl.ds(0, CHUNK)], rc6_v, sem_r).wait()

  def issue_scatters(s):
    for k in range(K):
      pltpu.make_async_copy(ev[s].at[pl.ds(k * ROW, ROW)],
                            e_sh.at[ein[s].at[k, 0]],
                            sem_s).start(add=True)

  def wait_scatters(s):
    pltpu.make_async_copy(out_hbm.at[cid, pl.ds(0, CHUNK)], ev[s],
                          sem_s).wait()

  def pair_compute(s):
    for g in range(CHUNK // 16):
      sl = pl.ds(g * 16, 16)
      r16 = lax.iota(jnp.int32, 16) + (g * 16)
      sp_s = plsc.bitcast(
          plsc.load_gather(nds[s], [r16, _col(6)]), jnp.int32)
      sp_d = plsc.bitcast(
          plsc.load_gather(ndd[s], [r16, _col(6)]), jnp.int32)
      pair_v[g // (ROW // 16), pl.ds((g % (ROW // 16)) * 16, 16)] = (
          sp_s * NZ + sp_d)

  def compute(s):
    for g in range(CHUNK // 16):
      sl = pl.ds(g * 16, 16)
      r16 = lax.iota(jnp.int32, 16) + (g * 16)
      k = g // (ROW // 16)
      gi = g % (ROW // 16)
      ws = [plsc.load_gather(nds[s], [r16, _col(a)]) for a in range(NREF)]
      wd = [plsc.load_gather(ndd[s], [r16, _col(b)]) for b in range(NREF)]
      gs16 = plsc.load_gather(nds[s], [r16, _col(NREF)])
      gd16 = plsc.load_gather(ndd[s], [r16, _col(NREF)])
      c6 = jnp.zeros((16,), jnp.float32)
      for a in range(NREF):
        for b in range(NREF):
          cab = plsc.load_gather(rc6_v, [r16, _col(a * NREF + b)])
          c6 = c6 + cab * ws[a] * wd[b]
      gg = gs16 * gd16
      qq = 3.0 * gg * gg
      r0d = (0.4 * SQRT3) * gg + 5.0
      dist = plsc.bitcast(ein[s][k, 2, pl.ds(gi * 16, 16)], jnp.float32)
      sw = plsc.bitcast(ein[s][k, 3, pl.ds(gi * 16, 16)], jnp.float32)
      rij = jnp.maximum(dist * INV_ANG, 1e-6)
      r2 = rij * rij
      r4 = r2 * r2
      r6 = r4 * r2
      r8 = r4 * r4
      p2 = r0d * r0d
      p4 = p2 * p2
      p6 = p4 * p2
      p8 = p4 * p4
      t6 = 1.0 / (r6 + p6)
      t8 = 1.0 / (r8 + p8)
      ev[s][sl] = (-0.5) * sw * (c6 * t6 + c6 * qq * t8)

  def batch(j, b, s, first, last):
    wait_gathers(s)
    pair_compute(s)
    issue_rc6()
    if first:
      @pl.when(j > 0)
      def _():
        wait_scatters(1 - s)
    else:
      wait_scatters(1 - s)

    def prefetch():
      lin_desc(b + 1, 1 - s).start()
      lin_desc(b + 1, 1 - s).wait()
      issue_gathers(1 - s)
    if last:
      @pl.when(j < (NBATCH // 2 - 1))
      def _():
        prefetch()
    else:
      prefetch()
    wait_rc6()
    compute(s)
    issue_scatters(s)

  lin_desc(0, 0).start()
  lin_desc(0, 0).wait()
  issue_gathers(0)

  @pl.loop(0, NBATCH // 2)
  def _(j):
    batch(j, j * 2, 0, True, False)
    batch(j, j * 2 + 1, 1, False, True)

  wait_scatters(1)

  plsc.subcore_barrier()
  pltpu.sync_copy(e_sh.at[sl_stage], ob_v)
  pltpu.sync_copy(ob_v, out_hbm.at[cid, sl_stage])


def _run_pass2(ein, nd, rc6p):
  fn = pl.kernel(
      _pass2_body,
      out_type=jax.ShapeDtypeStruct((2, NPAD), jnp.float32),
      mesh=_sc_mesh(),
      compiler_params=pltpu.CompilerParams(
          needs_layout_passes=False, use_tc_tiling_on_sc=False),
      scratch_types=[
          pltpu.VMEM_SHARED((NPAD,), jnp.float32),
          pltpu.VMEM((K, 4, ROW), jnp.int32),
          pltpu.VMEM((K, 4, ROW), jnp.int32),
          pltpu.VMEM((CHUNK, 8), jnp.float32),
          pltpu.VMEM((CHUNK, 8), jnp.float32),
          pltpu.VMEM((CHUNK, 8), jnp.float32),
          pltpu.VMEM((CHUNK, 8), jnp.float32),
          pltpu.VMEM((K, ROW), jnp.int32),
          pltpu.VMEM((CHUNK, RC6_W), jnp.float32),
          pltpu.VMEM((CHUNK,), jnp.float32),
          pltpu.VMEM((CHUNK,), jnp.float32),
          pltpu.VMEM((NSLICE,), jnp.float32),
          pltpu.SemaphoreType.DMA,
          pltpu.SemaphoreType.DMA,
          pltpu.SemaphoreType.DMA,
          pltpu.SemaphoreType.DMA,
      ],
  )
  return fn(ein, nd, rc6p)


def _final_body(e0_ref, e1_ref, out_ref):
  out_ref[...] = e0_ref[...] + e1_ref[...]


def _run_final(e0, e1):
  return pl.pallas_call(
      _final_body,
      grid=(NPAD // BLK,),
      in_specs=[
          pl.BlockSpec((BLK,), lambda i: (i,)),
          pl.BlockSpec((BLK,), lambda i: (i,)),
      ],
      out_specs=pl.BlockSpec((BLK,), lambda i: (i,)),
      out_shape=jax.ShapeDtypeStruct((NPAD,), jnp.float32),
  )(e0, e1)


@jax.jit
def kernel(species, edge_src, edge_dst, distances, switch,
           cov_d3, r4r2, ref_cn, ref_c6):
  sp_p = jnp.zeros((NPAD,), jnp.int32).at[:N_NODES].set(species)

  npad_extra = EPAD - N_EDGES
  pad_idx = (jnp.arange(npad_extra, dtype=jnp.int32) % (NPAD - N_NODES)
             ) + N_NODES
  es_p = jnp.concatenate([edge_src, pad_idx])
  ed_p = jnp.concatenate([edge_dst, pad_idx])
  d_p = jnp.concatenate(
      [distances, jnp.ones((npad_extra,), jnp.float32)])
  sw_p = jnp.concatenate(
      [switch, jnp.zeros((npad_extra,), jnp.float32)])
  ein = jnp.stack([
      es_p.reshape(NROWS, ROW),
      ed_p.reshape(NROWS, ROW),
      lax.bitcast_convert_type(d_p, jnp.int32).reshape(NROWS, ROW),
      lax.bitcast_convert_type(sw_p, jnp.int32).reshape(NROWS, ROW),
  ], axis=1)
  cov_p = jnp.zeros((96,), jnp.float32).at[:NZ].set(cov_d3)

  g = jnp.sqrt(r4r2)
  exw = jax.nn.one_hot(jnp.argmax(ref_cn, axis=1), NREF, dtype=jnp.float32)
  table = jnp.zeros((128, 128), jnp.float32)
  table = table.at[:NZ, 0:NREF].set(ref_cn)
  table = table.at[:NZ, NREF:2 * NREF].set(exw)
  table = table.at[:NZ, 2 * NREF].set(g)

  rc6p = jnp.zeros((RC6_PAD, RC6_W), jnp.float32)
  rc6p = rc6p.at[:RC6_ROWS, :NREF * NREF].set(
      ref_c6.reshape(RC6_ROWS, NREF * NREF))

  cnp = _run_pass1(sp_p, ein, cov_p)
  nd = _run_node(sp_p, cnp[0], cnp[1], table)
  ep = _run_pass2(ein, nd, rc6p)
  energy = _run_final(ep[0], ep[1])
  return energy[:N_NODES]


# speedup vs baseline: 1.5066x; 1.2573x over previous
import jax
import jax.numpy as jnp
from jax import lax
from jax.experimental import pallas as pl
from jax.experimental.pallas import tpu as pltpu
from jax.experimental.pallas import tpu_sc as plsc

ANG = 0.52917721067
INV_ANG = 1.0 / ANG
SQRT3 = 1.7320508075688772

N_NODES = 100000
NPAD = 102400
N_EDGES = 3200000
ROW = 80
K = 5
NWORK = 32
NROWS = 40000
EPAD = NROWS * ROW
CHUNK = K * ROW
NZ = 95
NREF = 5
RC6_ROWS = NZ * NZ
RC6_PAD = 9088
RC6_W = 16

ROWS_PER_W = NROWS // NWORK
NBATCH = ROWS_PER_W // K
NSLICE = NPAD // 16
BLK = 2048


def _zero_fill(ref, n):
  @pl.loop(0, n // 16)
  def _(i):
    ref[pl.ds(i * 16, 16)] = jnp.zeros((16,), jnp.float32)


def _sc_mesh():
  return plsc.VectorSubcoreMesh(core_axis_name="c", subcore_axis_name="s")


def _col(c):
  return jnp.full((16,), c, jnp.int32)


def _pass1_body(sp_hbm, ein_hbm, cov_hbm, out_hbm,
                cn_sh, cov_v,
                ein0_v, ein1_v, sps0_v, sps1_v, spd0_v, spd1_v,
                cn0_v, cn1_v, ob_v,
                sem_lin, sem_g, sem_s):
  cid = lax.axis_index("c")
  sid = lax.axis_index("s")
  wid = sid * 2 + cid

  sl_stage = pl.ds(sid * NSLICE, NSLICE)
  _zero_fill(ob_v, NSLICE)
  pltpu.sync_copy(ob_v, cn_sh.at[sl_stage])
  pltpu.sync_copy(cov_hbm, cov_v)
  plsc.subcore_barrier()

  r0 = wid * ROWS_PER_W
  ein = (ein0_v, ein1_v)
  sps = (sps0_v, sps1_v)
  spd = (spd0_v, spd1_v)
  cnb = (cn0_v, cn1_v)

  def lin_desc(b, s):
    return pltpu.make_async_copy(ein_hbm.at[pl.ds(r0 + b * K, K)],
                                 ein[s], sem_lin)

  def issue_gathers(s):
    for k in range(K):
      dst = pl.ds(k * ROW, ROW)
      pltpu.make_async_copy(sp_hbm.at[ein[s].at[k, 0]],
                            sps[s].at[dst], sem_g).start()
      pltpu.make_async_copy(sp_hbm.at[ein[s].at[k, 1]],
                            spd[s].at[dst], sem_g).start()

  def wait_gathers(s):
    pltpu.make_async_copy(sp_hbm.at[pl.ds(0, CHUNK)], sps[s], sem_g).wait()
    pltpu.make_async_copy(sp_hbm.at[pl.ds(0, CHUNK)], spd[s], sem_g).wait()

  def issue_scatters(s):
    for k in range(K):
      pltpu.make_async_copy(cnb[s].at[pl.ds(k * ROW, ROW)],
                            cn_sh.at[ein[s].at[k, 0]],
                            sem_s).start(add=True)

  def wait_scatters(s):
    pltpu.make_async_copy(out_hbm.at[cid, pl.ds(0, CHUNK)], cnb[s],
                          sem_s).wait()

  def compute(s):
    for g in range(CHUNK // 16):
      sl = pl.ds(g * 16, 16)
      k = g // (ROW // 16)
      gi = g % (ROW // 16)
      rc = (plsc.load_gather(cov_v, [sps[s][sl]]) +
            plsc.load_gather(cov_v, [spd[s][sl]]))
      dist = plsc.bitcast(ein[s][k, 2, pl.ds(gi * 16, 16)], jnp.float32)
      rij = jnp.maximum(dist * INV_ANG, 1e-6)
      x = 16.0 * (rc / rij - 1.0)
      cnb[s][sl] = 1.0 / (1.0 + jnp.exp(-x))

  def batch(j, b, s, first, last):
    wait_gathers(s)
    if first:
      @pl.when(j > 0)
      def _():
        wait_scatters(1 - s)
    else:
      wait_scatters(1 - s)

    def prefetch():
      lin_desc(b + 1, 1 - s).start()
      lin_desc(b + 1, 1 - s).wait()
      issue_gathers(1 - s)
    if last:
      @pl.when(j < (NBATCH // 2 - 1))
      def _():
        prefetch()
    else:
      prefetch()
    compute(s)
    issue_scatters(s)

  lin_desc(0, 0).start()
  lin_desc(0, 0).wait()
  issue_gathers(0)

  @pl.loop(0, NBATCH // 2)
  def _(j):
    batch(j, j * 2, 0, True, False)
    batch(j, j * 2 + 1, 1, False, True)

  wait_scatters(1)

  plsc.subcore_barrier()
  pltpu.sync_copy(cn_sh.at[sl_stage], ob_v)
  pltpu.sync_copy(ob_v, out_hbm.at[cid, sl_stage])


def _run_pass1(sp_p, ein, cov_p):
  fn = pl.kernel(
      _pass1_body,
      out_type=jax.ShapeDtypeStruct((2, NPAD), jnp.float32),
      mesh=_sc_mesh(),
      compiler_params=pltpu.CompilerParams(
          needs_layout_passes=False, use_tc_tiling_on_sc=False),
      scratch_types=[
          pltpu.VMEM_SHARED((NPAD,), jnp.float32),
          pltpu.VMEM((96,), jnp.float32),
          pltpu.VMEM((K, 4, ROW), jnp.int32),
          pltpu.VMEM((K, 4, ROW), jnp.int32),
          pltpu.VMEM((CHUNK,), jnp.int32),
          pltpu.VMEM((CHUNK,), jnp.int32),
          pltpu.VMEM((CHUNK,), jnp.int32),
          pltpu.VMEM((CHUNK,), jnp.int32),
          pltpu.VMEM((CHUNK,), jnp.float32),
          pltpu.VMEM((CHUNK,), jnp.float32),
          pltpu.VMEM((NSLICE,), jnp.float32),
          pltpu.SemaphoreType.DMA,
          pltpu.SemaphoreType.DMA,
          pltpu.SemaphoreType.DMA,
      ],
  )
  return fn(sp_p, ein, cov_p)


def _node_body(sp_ref, p0_ref, p1_ref, t_ref, nd_ref):
  sp = sp_ref[...]
  oh = (sp[:, None] == lax.broadcasted_iota(jnp.int32, (BLK, 128), 1)
        ).astype(jnp.float32)
  r = jnp.dot(oh, t_ref[...], preferred_element_type=jnp.float32)
  refcn = r[:, 0:NREF]
  exw = r[:, NREF:2 * NREF]
  g = r[:, 2 * NREF:2 * NREF + 1]
  cn = p0_ref[...] + p1_ref[...]
  mask = refcn >= 0.0
  dcn = refcn - cn[:, None]
  w = jnp.where(mask, jnp.exp(-4.0 * dcn * dcn), 0.0)
  norm = jnp.sum(w, axis=1, keepdims=True)
  wn = jnp.where(mask, w / jnp.maximum(norm, 1e-6), 0.0)
  wf = jnp.where(norm < 1e-6, exw, wn)
  spf = lax.bitcast_convert_type(sp, jnp.float32)[:, None]
  nd_ref[...] = jnp.concatenate(
      [wf, g, spf, jnp.zeros((BLK, 1), jnp.float32)], axis=1)


def _run_node(sp_p, p0, p1, table):
  return pl.pallas_call(
      _node_body,
      grid=(NPAD // BLK,),
      in_specs=[
          pl.BlockSpec((BLK,), lambda i: (i,)),
          pl.BlockSpec((BLK,), lambda i: (i,)),
          pl.BlockSpec((BLK,), lambda i: (i,)),
          pl.BlockSpec((128, 128), lambda i: (0, 0)),
      ],
      out_specs=pl.BlockSpec((BLK, 8), lambda i: (i, 0)),
      out_shape=jax.ShapeDtypeStruct((NPAD, 8), jnp.float32),
  )(sp_p, p0, p1, table)


def _pass2_body(ein_hbm, nd_hbm, rc6_hbm, out_hbm,
                e_sh,
                ein0_v, ein1_v, nds0_v, nds1_v, ndd0_v, ndd1_v,
                pair_v, rc6_v, e0_v, e1_v, ob_v,
                sem_lin, sem_g, sem_r, sem_s):
  cid = lax.axis_index("c")
  sid = lax.axis_index("s")
  wid = sid * 2 + cid

  sl_stage = pl.ds(sid * NSLICE, NSLICE)
  _zero_fill(ob_v, NSLICE)
  pltpu.sync_copy(ob_v, e_sh.at[sl_stage])
  plsc.subcore_barrier()

  r0 = wid * ROWS_PER_W
  ein = (ein0_v, ein1_v)
  nds = (nds0_v, nds1_v)
  ndd = (ndd0_v, ndd1_v)
  ev = (e0_v, e1_v)

  def lin_desc(b, s):
    return pltpu.make_async_copy(ein_hbm.at[pl.ds(r0 + b * K, K)],
                                 ein[s], sem_lin)

  def issue_gathers(s):
    for k in range(K):
      dst = pl.ds(k * ROW, ROW)
      pltpu.make_async_copy(nd_hbm.at[ein[s].at[k, 0]],
                            nds[s].at[dst], sem_g).start()
      pltpu.make_async_copy(nd_hbm.at[ein[s].at[k, 1]],
                            ndd[s].at[dst], sem_g).start()

  def wait_gathers(s):
    pltpu.make_async_copy(nd_hbm.at[pl.ds(0, CHUNK)], nds[s], sem_g).wait()
    pltpu.make_async_copy(nd_hbm.at[pl.ds(0, CHUNK)], ndd[s], sem_g).wait()

  def issue_rc6():
    for k in range(K):
      pltpu.make_async_copy(rc6_hbm.at[pair_v.at[k]],
                            rc6_v.at[pl.ds(k * ROW, ROW)], sem_r).start()

  def wait_rc6():
    pltpu.make_async_copy(rc6_hbm.at[pl.ds(0, CHUNK)], rc6_v, sem_r).wait()

  def issue_scatters(s):
    for k in range(K):
      pltpu.make_async_copy(ev[s].at[pl.ds(k * ROW, ROW)],
                            e_sh.at[ein[s].at[k, 0]],
                            sem_s).start(add=True)

  def wait_scatters(s):
    pltpu.make_async_copy(out_hbm.at[cid, pl.ds(0, CHUNK)], ev[s],
                          sem_s).wait()

  def pair_compute(s):
    for g in range(CHUNK // 16):
      sl = pl.ds(g * 16, 16)
      r16 = lax.iota(jnp.int32, 16) + (g * 16)
      sp_s = plsc.bitcast(
          plsc.load_gather(nds[s], [r16, _col(6)]), jnp.int32)
      sp_d = plsc.bitcast(
          plsc.load_gather(ndd[s], [r16, _col(6)]), jnp.int32)
      pair_v[g // (ROW // 16), pl.ds((g % (ROW // 16)) * 16, 16)] = (
          sp_s * NZ + sp_d)

  def compute(s):
    for g in range(CHUNK // 16):
      sl = pl.ds(g * 16, 16)
      r16 = lax.iota(jnp.int32, 16) + (g * 16)
      k = g // (ROW // 16)
      gi = g % (ROW // 16)
      ws = [plsc.load_gather(nds[s], [r16, _col(a)]) for a in range(NREF)]
      wd = [plsc.load_gather(ndd[s], [r16, _col(b)]) for b in range(NREF)]
      gs16 = plsc.load_gather(nds[s], [r16, _col(NREF)])
      gd16 = plsc.load_gather(ndd[s], [r16, _col(NREF)])
      c6 = jnp.zeros((16,), jnp.float32)
      for j in range(13):
        word = plsc.load_gather(rc6_v, [r16, _col(j)])
        m0 = 2 * j
        m1 = 2 * j + 1
        clo = plsc.bitcast(lax.shift_left(word, 16), jnp.float32)
        c6 = c6 + clo * ws[m0 // NREF] * wd[m0 % NREF]
        if m1 < NREF * NREF:
          chi = plsc.bitcast(
              lax.bitwise_and(word, jnp.int32(-65536)), jnp.float32)
          c6 = c6 + chi * ws[m1 // NREF] * wd[m1 % NREF]
      gg = gs16 * gd16
      qq = 3.0 * gg * gg
      r0d = (0.4 * SQRT3) * gg + 5.0
      dist = plsc.bitcast(ein[s][k, 2, pl.ds(gi * 16, 16)], jnp.float32)
      sw = plsc.bitcast(ein[s][k, 3, pl.ds(gi * 16, 16)], jnp.float32)
      rij = jnp.maximum(dist * INV_ANG, 1e-6)
      r2 = rij * rij
      r4 = r2 * r2
      r6 = r4 * r2
      r8 = r4 * r4
      p2 = r0d * r0d
      p4 = p2 * p2
      p6 = p4 * p2
      p8 = p4 * p4
      t6 = 1.0 / (r6 + p6)
      t8 = 1.0 / (r8 + p8)
      ev[s][sl] = (-0.5) * sw * (c6 * t6 + c6 * qq * t8)

  def batch(j, b, s, first, last):
    wait_gathers(s)
    pair_compute(s)
    issue_rc6()
    if first:
      @pl.when(j > 0)
      def _():
        wait_scatters(1 - s)
    else:
      wait_scatters(1 - s)

    def prefetch():
      lin_desc(b + 1, 1 - s).start()
      lin_desc(b + 1, 1 - s).wait()
      issue_gathers(1 - s)
    if last:
      @pl.when(j < (NBATCH // 2 - 1))
      def _():
        prefetch()
    else:
      prefetch()
    wait_rc6()
    compute(s)
    issue_scatters(s)

  lin_desc(0, 0).start()
  lin_desc(0, 0).wait()
  issue_gathers(0)

  @pl.loop(0, NBATCH // 2)
  def _(j):
    batch(j, j * 2, 0, True, False)
    batch(j, j * 2 + 1, 1, False, True)

  wait_scatters(1)

  plsc.subcore_barrier()
  pltpu.sync_copy(e_sh.at[sl_stage], ob_v)
  pltpu.sync_copy(ob_v, out_hbm.at[cid, sl_stage])


def _run_pass2(ein, nd, rc6p):
  fn = pl.kernel(
      _pass2_body,
      out_type=jax.ShapeDtypeStruct((2, NPAD), jnp.float32),
      mesh=_sc_mesh(),
      compiler_params=pltpu.CompilerParams(
          needs_layout_passes=False, use_tc_tiling_on_sc=False),
      scratch_types=[
          pltpu.VMEM_SHARED((NPAD,), jnp.float32),
          pltpu.VMEM((K, 4, ROW), jnp.int32),
          pltpu.VMEM((K, 4, ROW), jnp.int32),
          pltpu.VMEM((CHUNK, 8), jnp.float32),
          pltpu.VMEM((CHUNK, 8), jnp.float32),
          pltpu.VMEM((CHUNK, 8), jnp.float32),
          pltpu.VMEM((CHUNK, 8), jnp.float32),
          pltpu.VMEM((K, ROW), jnp.int32),
          pltpu.VMEM((CHUNK, RC6_W), jnp.int32),
          pltpu.VMEM((CHUNK,), jnp.float32),
          pltpu.VMEM((CHUNK,), jnp.float32),
          pltpu.VMEM((NSLICE,), jnp.float32),
          pltpu.SemaphoreType.DMA,
          pltpu.SemaphoreType.DMA,
          pltpu.SemaphoreType.DMA,
          pltpu.SemaphoreType.DMA,
      ],
  )
  return fn(ein, nd, rc6p)


def _final_body(e0_ref, e1_ref, out_ref):
  out_ref[...] = e0_ref[...] + e1_ref[...]


def _run_final(e0, e1):
  return pl.pallas_call(
      _final_body,
      grid=(NPAD // BLK,),
      in_specs=[
          pl.BlockSpec((BLK,), lambda i: (i,)),
          pl.BlockSpec((BLK,), lambda i: (i,)),
      ],
      out_specs=pl.BlockSpec((BLK,), lambda i: (i,)),
      out_shape=jax.ShapeDtypeStruct((NPAD,), jnp.float32),
  )(e0, e1)


@jax.jit
def kernel(species, edge_src, edge_dst, distances, switch,
           cov_d3, r4r2, ref_cn, ref_c6):
  sp_p = jnp.zeros((NPAD,), jnp.int32).at[:N_NODES].set(species)

  npad_extra = EPAD - N_EDGES
  pad_idx = (jnp.arange(npad_extra, dtype=jnp.int32) % (NPAD - N_NODES)
             ) + N_NODES
  es_p = jnp.concatenate([edge_src, pad_idx])
  ed_p = jnp.concatenate([edge_dst, pad_idx])
  d_p = jnp.concatenate(
      [distances, jnp.ones((npad_extra,), jnp.float32)])
  sw_p = jnp.concatenate(
      [switch, jnp.zeros((npad_extra,), jnp.float32)])
  ein = jnp.stack([
      es_p.reshape(NROWS, ROW),
      ed_p.reshape(NROWS, ROW),
      lax.bitcast_convert_type(d_p, jnp.int32).reshape(NROWS, ROW),
      lax.bitcast_convert_type(sw_p, jnp.int32).reshape(NROWS, ROW),
  ], axis=1)
  cov_p = jnp.zeros((96,), jnp.float32).at[:NZ].set(cov_d3)

  g = jnp.sqrt(r4r2)
  exw = jax.nn.one_hot(jnp.argmax(ref_cn, axis=1), NREF, dtype=jnp.float32)
  table = jnp.zeros((128, 128), jnp.float32)
  table = table.at[:NZ, 0:NREF].set(ref_cn)
  table = table.at[:NZ, NREF:2 * NREF].set(exw)
  table = table.at[:NZ, 2 * NREF].set(g)

  c6flat = jnp.zeros((RC6_ROWS, 26), jnp.float32).at[:, :25].set(
      ref_c6.reshape(RC6_ROWS, 25)).astype(jnp.bfloat16)
  c6u16 = lax.bitcast_convert_type(c6flat, jnp.uint16).astype(jnp.uint32)
  c6words = (c6u16[:, 0::2] | (c6u16[:, 1::2] << 16)).astype(jnp.int32)
  rc6p = jnp.zeros((RC6_PAD, RC6_W), jnp.int32)
  rc6p = rc6p.at[:RC6_ROWS, :13].set(c6words)

  cnp = _run_pass1(sp_p, ein, cov_p)
  nd = _run_node(sp_p, cnp[0], cnp[1], table)
  ep = _run_pass2(ein, nd, rc6p)
  energy = _run_final(ep[0], ep[1])
  return energy[:N_NODES]
